# baseline (speedup 1.0000x reference)
import functools

import jax
import jax.numpy as jnp
from jax import lax
from jax.experimental import pallas as pl
from jax.experimental.pallas import tpu as pltpu
from jax.experimental.pallas import tpu_sc as plsc

OUT = 7
NSAMP = OUT * OUT
C = 128
NC, NS, L = 2, 16, 16
NW = NC * NS
N_BOX = 1000
BOX_PAD = 1024
BPW = BOX_PAD // NW

B = 2
LVL_H = [200, 100, 50, 25]
LVL_SCALE = [0.25, 0.125, 0.0625, 0.03125]

_T = [(224.0 * (2.0 ** (k - 4) - 1e-6)) ** 2 for k in (3, 4, 5)]
T2, T3, T4 = (float(t) for t in _T)

NBRS = 4
SSTR = 56
GLEN = 56
SLOT = NBRS * SSTR


def _mxu_t(x2d):
    eye = jnp.eye(C, dtype=jnp.float32)
    return lax.dot_general(x2d, eye, (((0,), (0,)), ((), ())),
                           preferred_element_type=jnp.float32)


def _tabs_body(f0, f1, f2, f3, o0, o1, o2, o3):
    st = pl.program_id(0)

    @pl.when(st < 10)
    def _():
        o0[...] = _mxu_t(f0[0].reshape(C, 8000))

    @pl.when((st >= 10) & (st < 12))
    def _():
        o1[...] = _mxu_t(f1[0].reshape(C, 10000))

    @pl.when(st == 12)
    def _():
        for bb in range(B):
            o2[pl.ds(bb * 2500, 2500), :] = _mxu_t(f2[bb].reshape(C, 2500))

    @pl.when(st == 13)
    def _():
        for bb in range(B):
            o3[pl.ds(bb * 625, 625), :] = _mxu_t(f3[bb].reshape(C, 625))


def _build_tables(feat0, feat1, feat2, feat3):
    c10 = lambda st: jnp.clip(st, 0, 9)
    return pl.pallas_call(
        _tabs_body,
        out_shape=[jax.ShapeDtypeStruct((2 * h * h, C), jnp.float32)
                   for h in LVL_H],
        grid=(14,),
        in_specs=[
            pl.BlockSpec((1, C, 40, 200), lambda st: (c10(st) // 5, 0,
                                                      c10(st) % 5, 0)),
            pl.BlockSpec((1, C, 100, 100), lambda st: (jnp.clip(st - 10, 0, 1),
                                                       0, 0, 0)),
            pl.BlockSpec((B, C, 50, 50), lambda st: (0, 0, 0, 0)),
            pl.BlockSpec((B, C, 25, 25), lambda st: (0, 0, 0, 0)),
        ],
        out_specs=[
            pl.BlockSpec((8000, C), lambda st: (c10(st), 0)),
            pl.BlockSpec((10000, C), lambda st: (jnp.clip(st - 10, 0, 1), 0)),
            pl.BlockSpec((2 * 2500, C), lambda st: (0, 0)),
            pl.BlockSpec((2 * 625, C), lambda st: (0, 0)),
        ],
    )(feat0, feat1, feat2, feat3)


def _pool_body(t0, t1, t2, t3, boxes_w, bidx_w, out, bx_v, bb_v, pf, pi,
               idx_v, w_v, rows_a, rows_b, out_ta, out_tb, sem_a, sem_b,
               sem_o):
    wid = lax.axis_index("s") * NC + lax.axis_index("c")
    sems = (sem_a, sem_b)
    rowsr = (rows_a, rows_b)
    tabs = (t0, t1, t2, t3)

    pltpu.sync_copy(boxes_w.at[wid], bx_v)
    pltpu.sync_copy(bidx_w.at[wid], bb_v)

    for k in range(BPW // L):
        lanes = pl.ds(k * L, L)
        x1 = bx_v[0, lanes]
        y1 = bx_v[1, lanes]
        x2 = bx_v[2, lanes]
        y2 = bx_v[3, lanes]
        b = bb_v[lanes]
        area = (x2 - x1 + 1.0) * (y2 - y1 + 1.0)
        one = jnp.full((L,), 1, jnp.int32)
        zero = jnp.full((L,), 0, jnp.int32)
        lv = (jnp.where(area >= T2, one, zero)
              + jnp.where(area >= T3, one, zero)
              + jnp.where(area >= T4, one, zero))
        scale = jnp.where(lv == 0, LVL_SCALE[0],
                          jnp.where(lv == 1, LVL_SCALE[1],
                                    jnp.where(lv == 2, LVL_SCALE[2],
                                              LVL_SCALE[3]))).astype(jnp.float32)
        hf = jnp.where(lv == 0, float(LVL_H[0]),
                       jnp.where(lv == 1, float(LVL_H[1]),
                                 jnp.where(lv == 2, float(LVL_H[2]),
                                           float(LVL_H[3])))).astype(jnp.float32)
        wi = hf.astype(jnp.int32)
        base = b * (wi * wi)
        x1s = x1 * scale
        y1s = y1 * scale
        binw = jnp.maximum(x2 * scale - x1s, 1.0) / float(OUT)
        binh = jnp.maximum(y2 * scale - y1s, 1.0) / float(OUT)
        pf[pl.ds(0 * BPW + k * L, L)] = x1s
        pf[pl.ds(1 * BPW + k * L, L)] = y1s
        pf[pl.ds(2 * BPW + k * L, L)] = binw
        pf[pl.ds(3 * BPW + k * L, L)] = binh
        pf[pl.ds(4 * BPW + k * L, L)] = hf
        pi[pl.ds(0 * BPW + k * L, L)] = base
        pi[pl.ds(1 * BPW + k * L, L)] = wi
        pi[pl.ds(2 * BPW + k * L, L)] = lv

    def compute_a(bi, rb):
        bvec = jnp.full((L,), 0, jnp.int32) + bi

        def pfrow(r):
            return plsc.load_gather(pf, [bvec + (r * BPW)])

        def pirow(r):
            return plsc.load_gather(pi, [bvec + (r * BPW)])

        x1s = pfrow(0)
        y1s = pfrow(1)
        binw = pfrow(2)
        binh = pfrow(3)
        hf = pfrow(4)
        base = pirow(0)
        wi = pirow(1)
        sbase = rb * SLOT

        for v in range(4):
            s = lax.iota(jnp.int32, 16) + (16 * v)
            ib = s // OUT
            jb = s - ib * OUT
            gy = y1s + (ib.astype(jnp.float32) + 0.5) * binh
            gx = x1s + (jb.astype(jnp.float32) + 0.5) * binw
            valid = ((gy > -1.0) & (gy < hf) & (gx > -1.0) & (gx < hf))
            y = jnp.minimum(jnp.maximum(gy, 0.0), hf - 1.0)
            x = jnp.minimum(jnp.maximum(gx, 0.0), hf - 1.0)
            yl = y.astype(jnp.int32)
            xl = x.astype(jnp.int32)
            ly = y - yl.astype(jnp.float32)
            lx = x - xl.astype(jnp.float32)
            hy = 1.0 - ly
            hx = 1.0 - lx
            vf = jnp.where(valid, jnp.full((L,), 1.0, jnp.float32),
                           jnp.full((L,), 0.0, jnp.float32))
            ws = (hy * hx * vf, hy * lx * vf, ly * hx * vf, ly * lx * vf)
            rowb = base + yl * wi + xl
            dx = jnp.minimum(xl + 1, wi - 1) - xl
            yh = jnp.minimum(yl + 1, wi - 1)
            row2 = base + yh * wi + xl
            ids = (rowb, rowb + dx, row2, row2 + dx)
            if v < 3:
                for n in range(NBRS):
                    idx_v[rb * 2 + n // 2,
                          pl.ds((n % 2) * SSTR + 16 * v, 16)] = ids[n]
                    w_v[pl.ds(sbase + n * SSTR + 16 * v, 16)] = ws[n]
            else:
                io = lax.iota(jnp.int32, 16)
                m = io < 8
                io8 = jnp.minimum(io, 7)
                row = jnp.full((L,), 0, jnp.int32)
                for n in range(NBRS):
                    plsc.store_scatter(idx_v, [row + (rb * 2 + n // 2),
                                               io8 + ((n % 2) * SSTR + 48)],
                                       ids[n], mask=m)
                    plsc.store_scatter(w_v, [io8 + (sbase + n * SSTR + 48)],
                                       ws[n], mask=m)

    def fire_box(rb, sem, lv0):
        for lvl in range(4):
            @pl.when(lv0 == lvl)
            def _(lvl=lvl):
                tab = tabs[lvl]
                for h in (0, 1):
                    pltpu.async_copy(tab.at[idx_v.at[rb * 2 + h]],
                                     rowsr[rb].at[h], sem)

    def drain_box(rb, sem, lv0):
        for lvl in range(4):
            @pl.when(lv0 == lvl)
            def _(lvl=lvl):
                tab = tabs[lvl]
                for h in (0, 1):
                    pltpu.make_async_copy(tab.at[idx_v.at[rb * 2 + h]],
                                          rowsr[rb].at[h], sem).wait()

    posk = tuple((lax.iota(jnp.int32, 16) + (16 * kk)) * NSAMP
                 for kk in range(C // L))

    def combine(bi, rb, ot):
        sbase = rb * SLOT
        rv = rowsr[rb]

        @plsc.parallel_loop(0, OUT, unroll=7)
        def cbody(q):
            for t in range(OUT):
                si = q * OUT + t
                svec = jnp.full((L,), 0, jnp.int32) + si
                w1 = plsc.load_gather(w_v, [svec + (sbase + 0 * SSTR)])
                w2 = plsc.load_gather(w_v, [svec + (sbase + 1 * SSTR)])
                w3 = plsc.load_gather(w_v, [svec + (sbase + 2 * SSTR)])
                w4 = plsc.load_gather(w_v, [svec + (sbase + 3 * SSTR)])
                for kk in range(C // L):
                    cl = pl.ds(16 * kk, 16)
                    r1 = rv[0, si, cl]
                    r2 = rv[0, SSTR + si, cl]
                    r3 = rv[1, si, cl]
                    r4 = rv[1, SSTR + si, cl]
                    acc = (w1 * r1 + w2 * r2) + (w3 * r3 + w4 * r4)
                    plsc.store_scatter(ot, [posk[kk] + si], acc)

    outr = (out_ta, out_tb)

    def pair_body(g, carry):
        for b2 in (0, 1):
            bi = 2 * g + b2
            compute_a(bi, b2)
            lvv = plsc.load_gather(pi, [jnp.full((L,), 2 * BPW, jnp.int32)
                                        + bi])
            fire_box(b2, sems[b2], lvv[0])
            drain_box(b2, sems[b2], lvv[0])

            @pl.when(bi >= 2)
            def _():
                pltpu.make_async_copy(outr[b2], out.at[wid * BPW],
                                      sem_o).wait()

            combine(bi, b2, outr[b2])
            pltpu.async_copy(outr[b2], out.at[wid * BPW + bi], sem_o)
        return carry

    lax.fori_loop(0, BPW // 2, pair_body, 0)
    for b2 in (0, 1):
        pltpu.make_async_copy(outr[b2], out.at[wid * BPW], sem_o).wait()


@jax.jit
def _sc_pool(t0, t1, t2, t3, boxes_w, bidx_w):
    mesh = plsc.VectorSubcoreMesh(core_axis_name="c", subcore_axis_name="s",
                                  num_cores=NC, num_subcores=NS)
    f = functools.partial(
        pl.kernel,
        out_type=jax.ShapeDtypeStruct((BOX_PAD, C * NSAMP), jnp.float32),
        mesh=mesh,
        compiler_params=pltpu.CompilerParams(needs_layout_passes=False),
        scratch_types=[
            pltpu.VMEM((4, BPW), jnp.float32),
            pltpu.VMEM((BPW,), jnp.int32),
            pltpu.VMEM((8 * BPW,), jnp.float32),
            pltpu.VMEM((3 * BPW,), jnp.int32),
            pltpu.VMEM((2 * 2, 2 * GLEN), jnp.int32),
            pltpu.VMEM((2 * SLOT,), jnp.float32),
            pltpu.VMEM((2, 2 * GLEN, C), jnp.float32),
            pltpu.VMEM((2, 2 * GLEN, C), jnp.float32),
            pltpu.VMEM((C * NSAMP,), jnp.float32),
            pltpu.VMEM((C * NSAMP,), jnp.float32),
            pltpu.SemaphoreType.DMA,
            pltpu.SemaphoreType.DMA,
            pltpu.SemaphoreType.DMA,
        ],
    )(_pool_body)
    return f(t0, t1, t2, t3, boxes_w, bidx_w)


def kernel(feat0, feat1, feat2, feat3, boxes, batch_idx):
    tables = _build_tables(feat0, feat1, feat2, feat3)
    boxes_p = jnp.pad(boxes, ((0, BOX_PAD - N_BOX), (0, 0)))
    boxes_w = boxes_p.T.reshape(4, NW, BPW).transpose(1, 0, 2)
    bidx_w = jnp.pad(batch_idx.astype(jnp.int32),
                     (0, BOX_PAD - N_BOX)).reshape(NW, BPW)
    out = _sc_pool(*tables, boxes_w, bidx_w)
    return out[:N_BOX].reshape(N_BOX, C, OUT, OUT)


# speedup vs baseline: 1.2201x; 1.2201x over previous
import functools

import jax
import jax.numpy as jnp
from jax import lax
from jax.experimental import pallas as pl
from jax.experimental.pallas import tpu as pltpu
from jax.experimental.pallas import tpu_sc as plsc

OUT = 7
NSAMP = OUT * OUT
C = 128
NC, NS, L = 2, 16, 16
NW = NC * NS
N_BOX = 1000
BOX_PAD = 1024
BPW = BOX_PAD // NW

B = 2
LVL_H = [200, 100, 50, 25]
LVL_SCALE = [0.25, 0.125, 0.0625, 0.03125]

_T = [(224.0 * (2.0 ** (k - 4) - 1e-6)) ** 2 for k in (3, 4, 5)]
T2, T3, T4 = (float(t) for t in _T)

NBRS = 4
SSTR = 56
GLEN = 56
SLOT = NBRS * SSTR


def _mxu_t(x2d):
    eye = jnp.eye(C, dtype=jnp.float32)
    return lax.dot_general(x2d, eye, (((0,), (0,)), ((), ())),
                           preferred_element_type=jnp.float32)


def _tabs_body(f0, f1, f2, f3, o0, o1, o2, o3):
    st = pl.program_id(0)

    @pl.when(st < 10)
    def _():
        o0[...] = _mxu_t(f0[0].reshape(C, 8000))

    @pl.when((st >= 10) & (st < 12))
    def _():
        o1[...] = _mxu_t(f1[0].reshape(C, 10000))

    @pl.when(st == 12)
    def _():
        for bb in range(B):
            o2[pl.ds(bb * 2500, 2500), :] = _mxu_t(f2[bb].reshape(C, 2500))

    @pl.when(st == 13)
    def _():
        for bb in range(B):
            o3[pl.ds(bb * 625, 625), :] = _mxu_t(f3[bb].reshape(C, 625))


def _build_tables(feat0, feat1, feat2, feat3):
    c10 = lambda st: jnp.clip(st, 0, 9)
    return pl.pallas_call(
        _tabs_body,
        out_shape=[jax.ShapeDtypeStruct((2 * h * h, C), jnp.float32)
                   for h in LVL_H],
        grid=(14,),
        in_specs=[
            pl.BlockSpec((1, C, 40, 200), lambda st: (c10(st) // 5, 0,
                                                      c10(st) % 5, 0)),
            pl.BlockSpec((1, C, 100, 100), lambda st: (jnp.clip(st - 10, 0, 1),
                                                       0, 0, 0)),
            pl.BlockSpec((B, C, 50, 50), lambda st: (0, 0, 0, 0)),
            pl.BlockSpec((B, C, 25, 25), lambda st: (0, 0, 0, 0)),
        ],
        out_specs=[
            pl.BlockSpec((8000, C), lambda st: (c10(st), 0)),
            pl.BlockSpec((10000, C), lambda st: (jnp.clip(st - 10, 0, 1), 0)),
            pl.BlockSpec((2 * 2500, C), lambda st: (0, 0)),
            pl.BlockSpec((2 * 625, C), lambda st: (0, 0)),
        ],
    )(feat0, feat1, feat2, feat3)


def _pool_body(t0, t1, t2, t3, boxes_w, bidx_w, out, bx_v, bb_v, pf, pi,
               idx_v, w_v, rows_a, rows_b, out_ta, out_tb, sem_a, sem_b,
               sem_o):
    wid = lax.axis_index("s") * NC + lax.axis_index("c")
    sems = (sem_a, sem_b)
    rowsr = (rows_a, rows_b)
    tabs = (t0, t1, t2, t3)

    pltpu.sync_copy(boxes_w.at[wid], bx_v)
    pltpu.sync_copy(bidx_w.at[wid], bb_v)

    for k in range(BPW // L):
        lanes = pl.ds(k * L, L)
        x1 = bx_v[0, lanes]
        y1 = bx_v[1, lanes]
        x2 = bx_v[2, lanes]
        y2 = bx_v[3, lanes]
        b = bb_v[lanes]
        area = (x2 - x1 + 1.0) * (y2 - y1 + 1.0)
        one = jnp.full((L,), 1, jnp.int32)
        zero = jnp.full((L,), 0, jnp.int32)
        lv = (jnp.where(area >= T2, one, zero)
              + jnp.where(area >= T3, one, zero)
              + jnp.where(area >= T4, one, zero))
        scale = jnp.where(lv == 0, LVL_SCALE[0],
                          jnp.where(lv == 1, LVL_SCALE[1],
                                    jnp.where(lv == 2, LVL_SCALE[2],
                                              LVL_SCALE[3]))).astype(jnp.float32)
        hf = jnp.where(lv == 0, float(LVL_H[0]),
                       jnp.where(lv == 1, float(LVL_H[1]),
                                 jnp.where(lv == 2, float(LVL_H[2]),
                                           float(LVL_H[3])))).astype(jnp.float32)
        wi = hf.astype(jnp.int32)
        base = b * (wi * wi)
        x1s = x1 * scale
        y1s = y1 * scale
        binw = jnp.maximum(x2 * scale - x1s, 1.0) / float(OUT)
        binh = jnp.maximum(y2 * scale - y1s, 1.0) / float(OUT)
        pf[pl.ds(0 * BPW + k * L, L)] = x1s
        pf[pl.ds(1 * BPW + k * L, L)] = y1s
        pf[pl.ds(2 * BPW + k * L, L)] = binw
        pf[pl.ds(3 * BPW + k * L, L)] = binh
        pf[pl.ds(4 * BPW + k * L, L)] = hf
        pi[pl.ds(0 * BPW + k * L, L)] = base
        pi[pl.ds(1 * BPW + k * L, L)] = wi
        pi[pl.ds(2 * BPW + k * L, L)] = lv

    def compute_a(bi, rb):
        bvec = jnp.full((L,), 0, jnp.int32) + bi

        def pfrow(r):
            return plsc.load_gather(pf, [bvec + (r * BPW)])

        def pirow(r):
            return plsc.load_gather(pi, [bvec + (r * BPW)])

        x1s = pfrow(0)
        y1s = pfrow(1)
        binw = pfrow(2)
        binh = pfrow(3)
        hf = pfrow(4)
        base = pirow(0)
        wi = pirow(1)
        sbase = rb * SLOT

        for v in range(4):
            s = lax.iota(jnp.int32, 16) + (16 * v)
            ib = s // OUT
            jb = s - ib * OUT
            gy = y1s + (ib.astype(jnp.float32) + 0.5) * binh
            gx = x1s + (jb.astype(jnp.float32) + 0.5) * binw
            valid = ((gy > -1.0) & (gy < hf) & (gx > -1.0) & (gx < hf))
            y = jnp.minimum(jnp.maximum(gy, 0.0), hf - 1.0)
            x = jnp.minimum(jnp.maximum(gx, 0.0), hf - 1.0)
            yl = y.astype(jnp.int32)
            xl = x.astype(jnp.int32)
            ly = y - yl.astype(jnp.float32)
            lx = x - xl.astype(jnp.float32)
            hy = 1.0 - ly
            hx = 1.0 - lx
            vf = jnp.where(valid, jnp.full((L,), 1.0, jnp.float32),
                           jnp.full((L,), 0.0, jnp.float32))
            ws = (hy * hx * vf, hy * lx * vf, ly * hx * vf, ly * lx * vf)
            rowb = base + yl * wi + xl
            dx = jnp.minimum(xl + 1, wi - 1) - xl
            yh = jnp.minimum(yl + 1, wi - 1)
            row2 = base + yh * wi + xl
            ids = (rowb, rowb + dx, row2, row2 + dx)
            if v < 3:
                for n in range(NBRS):
                    idx_v[rb * 2 + n // 2,
                          pl.ds((n % 2) * SSTR + 16 * v, 16)] = ids[n]
                    w_v[pl.ds(sbase + n * SSTR + 16 * v, 16)] = ws[n]
            else:
                io = lax.iota(jnp.int32, 16)
                m = io < 8
                io8 = jnp.minimum(io, 7)
                row = jnp.full((L,), 0, jnp.int32)
                for n in range(NBRS):
                    plsc.store_scatter(idx_v, [row + (rb * 2 + n // 2),
                                               io8 + ((n % 2) * SSTR + 48)],
                                       ids[n], mask=m)
                    plsc.store_scatter(w_v, [io8 + (sbase + n * SSTR + 48)],
                                       ws[n], mask=m)

    def fire_box(rb, sem, lv0):
        for lvl in range(4):
            @pl.when(lv0 == lvl)
            def _(lvl=lvl):
                tab = tabs[lvl]
                for h in (0, 1):
                    pltpu.async_copy(tab.at[idx_v.at[rb * 2 + h]],
                                     rowsr[rb].at[h], sem)

    def drain_box(rb, sem, lv0):
        for lvl in range(4):
            @pl.when(lv0 == lvl)
            def _(lvl=lvl):
                tab = tabs[lvl]
                for h in (0, 1):
                    pltpu.make_async_copy(tab.at[idx_v.at[rb * 2 + h]],
                                          rowsr[rb].at[h], sem).wait()

    posk = tuple((lax.iota(jnp.int32, 16) + (16 * kk)) * NSAMP
                 for kk in range(C // L))

    def combine(bi, rb, ot):
        sbase = rb * SLOT
        rv = rowsr[rb]

        @plsc.parallel_loop(0, OUT)
        def cbody(q):
            for t in range(OUT):
                si = q * OUT + t
                svec = jnp.full((L,), 0, jnp.int32) + si
                w1 = plsc.load_gather(w_v, [svec + (sbase + 0 * SSTR)])
                w2 = plsc.load_gather(w_v, [svec + (sbase + 1 * SSTR)])
                w3 = plsc.load_gather(w_v, [svec + (sbase + 2 * SSTR)])
                w4 = plsc.load_gather(w_v, [svec + (sbase + 3 * SSTR)])
                for kk in range(C // L):
                    cl = pl.ds(16 * kk, 16)
                    r1 = rv[0, si, cl]
                    r2 = rv[0, SSTR + si, cl]
                    r3 = rv[1, si, cl]
                    r4 = rv[1, SSTR + si, cl]
                    acc = (w1 * r1 + w2 * r2) + (w3 * r3 + w4 * r4)
                    plsc.store_scatter(ot, [posk[kk] + si], acc)

    outr = (out_ta, out_tb)

    def pair_body(g, carry):
        for b2 in (0, 1):
            bi = 2 * g + b2
            compute_a(bi, b2)
            lvv = plsc.load_gather(pi, [jnp.full((L,), 2 * BPW, jnp.int32)
                                        + bi])
            fire_box(b2, sems[b2], lvv[0])
            drain_box(b2, sems[b2], lvv[0])

            @pl.when(bi >= 2)
            def _():
                pltpu.make_async_copy(outr[b2], out.at[wid * BPW],
                                      sem_o).wait()

            combine(bi, b2, outr[b2])
            pltpu.async_copy(outr[b2], out.at[wid * BPW + bi], sem_o)
        return carry

    lax.fori_loop(0, BPW // 2, pair_body, 0)
    for b2 in (0, 1):
        pltpu.make_async_copy(outr[b2], out.at[wid * BPW], sem_o).wait()


@jax.jit
def _sc_pool(t0, t1, t2, t3, boxes_w, bidx_w):
    mesh = plsc.VectorSubcoreMesh(core_axis_name="c", subcore_axis_name="s",
                                  num_cores=NC, num_subcores=NS)
    f = functools.partial(
        pl.kernel,
        out_type=jax.ShapeDtypeStruct((BOX_PAD, C * NSAMP), jnp.float32),
        mesh=mesh,
        compiler_params=pltpu.CompilerParams(needs_layout_passes=False),
        scratch_types=[
            pltpu.VMEM((4, BPW), jnp.float32),
            pltpu.VMEM((BPW,), jnp.int32),
            pltpu.VMEM((8 * BPW,), jnp.float32),
            pltpu.VMEM((3 * BPW,), jnp.int32),
            pltpu.VMEM((2 * 2, 2 * GLEN), jnp.int32),
            pltpu.VMEM((2 * SLOT,), jnp.float32),
            pltpu.VMEM((2, 2 * GLEN, C), jnp.float32),
            pltpu.VMEM((2, 2 * GLEN, C), jnp.float32),
            pltpu.VMEM((C * NSAMP,), jnp.float32),
            pltpu.VMEM((C * NSAMP,), jnp.float32),
            pltpu.SemaphoreType.DMA,
            pltpu.SemaphoreType.DMA,
            pltpu.SemaphoreType.DMA,
        ],
    )(_pool_body)
    return f(t0, t1, t2, t3, boxes_w, bidx_w)


def kernel(feat0, feat1, feat2, feat3, boxes, batch_idx):
    tables = _build_tables(feat0, feat1, feat2, feat3)
    boxes_p = jnp.pad(boxes, ((0, BOX_PAD - N_BOX), (0, 0)))
    boxes_w = boxes_p.T.reshape(4, NW, BPW).transpose(1, 0, 2)
    bidx_w = jnp.pad(batch_idx.astype(jnp.int32),
                     (0, BOX_PAD - N_BOX)).reshape(NW, BPW)
    out = _sc_pool(*tables, boxes_w, bidx_w)
    return out[:N_BOX].reshape(N_BOX, C, OUT, OUT)
